# Initial kernel scaffold; baseline (speedup 1.0000x reference)
import jax
import jax.numpy as jnp
from jax.experimental import pallas as pl


def kernel(x, edge_index, W1, b1, g1, be1, W2, b2, g2, be2, W3, b3):
    raise NotImplementedError("write your pallas kernel here")


# speedup vs baseline: 13.7509x; 13.7509x over previous
import functools

import jax
import jax.numpy as jnp
from jax import lax
from jax.experimental import pallas as pl
from jax.experimental.pallas import tpu as pltpu
from jax.experimental.pallas import tpu_sc as plsc

NN = 10000
EE = 320000
HID = 128
NCLS = 10
EPSV = 1e-5

NP_ = 10240
NCORE = 2
NSUB = 16
NWORK = NCORE * NSUB
K = 128
NCH = 80
NCH2 = 160
EPAD = NWORK * NCH * K
ROWS_T = NP_ // NSUB


def _sc_mesh():
    return plsc.VectorSubcoreMesh(
        core_axis_name="c", subcore_axis_name="s",
        num_cores=NCORE, num_subcores=NSUB)


def _deg_body(dst_hbm, deg_hbm, dst_v, ones_v, z_v, deg_sh, sem):
    del sem
    c = lax.axis_index("c")
    s = lax.axis_index("s")
    w = s * NCORE + c
    for i in range(K // 16):
        ones_v[pl.ds(i * 16, 16)] = jnp.ones((16,), jnp.float32)
    for i in range(ROWS_T // 16):
        z_v[pl.ds(i * 16, 16)] = jnp.zeros((16,), jnp.float32)
    pltpu.sync_copy(z_v, deg_sh.at[pl.ds(s * ROWS_T, ROWS_T)])
    pltpu.sync_copy(dst_hbm.at[w], dst_v)
    plsc.subcore_barrier()

    def body(j, carry):
        pltpu.sync_copy(ones_v, deg_sh.at[dst_v.at[j]], add=True)
        return carry

    lax.fori_loop(0, NCH, body, 0)
    plsc.subcore_barrier()
    pltpu.sync_copy(deg_sh.at[pl.ds(s * ROWS_T, ROWS_T)],
                    deg_hbm.at[c, pl.ds(s * ROWS_T, ROWS_T)])


_deg_call = pl.kernel(
    _deg_body,
    out_type=jax.ShapeDtypeStruct((NCORE, NP_), jnp.float32),
    mesh=_sc_mesh(),
    compiler_params=pltpu.CompilerParams(use_tc_tiling_on_sc=False),
    scratch_types=[
        pltpu.VMEM((NCH, K), jnp.int32),
        pltpu.VMEM((K,), jnp.float32),
        pltpu.VMEM((ROWS_T,), jnp.float32),
        pltpu.VMEM_SHARED((NP_,), jnp.float32),
        pltpu.SemaphoreType.DMA,
    ],
)


def _edge_split_body(y_hbm, src_hbm, dst_hbm, acc_hbm, src_v, dst_v, rows_v,
                     acc_sh, g0, g1):
    c = lax.axis_index("c")
    s = lax.axis_index("s")
    D = 64
    for q in range(ROWS_T // K):
        pltpu.sync_copy(y_hbm.at[pl.ds(NN, K)],
                        acc_sh.at[pl.ds(s * ROWS_T + q * K, K)])
    pltpu.sync_copy(src_hbm.at[c, s], src_v)
    pltpu.sync_copy(dst_hbm.at[s], dst_v)
    plsc.subcore_barrier()

    def fire(j, b):
        sem = g0 if b == 0 else g1
        pltpu.async_copy(y_hbm.at[src_v.at[j]], rows_v.at[b], sem)

    def wait(b):
        sem = g0 if b == 0 else g1
        pltpu.make_async_copy(y_hbm.at[pl.ds(0, K)], rows_v.at[b], sem).wait()

    def scat(j, b):
        pltpu.sync_copy(rows_v.at[b], acc_sh.at[dst_v.at[j]], add=True)

    fire(0, 0)
    fire(1, 1)

    def body_i(i, carry):
        j0 = 2 * i
        wait(0)
        scat(j0, 0)
        fire(jnp.minimum(j0 + 2, NCH2 - 1), 0)
        wait(1)
        scat(j0 + 1, 1)
        fire(jnp.minimum(j0 + 3, NCH2 - 1), 1)
        return carry

    lax.fori_loop(0, NCH2 // 2, body_i, 0)
    wait(0)
    wait(1)
    plsc.subcore_barrier()
    pltpu.sync_copy(acc_sh.at[pl.ds(s * ROWS_T, ROWS_T)],
                    acc_hbm.at[c, pl.ds(s * ROWS_T, ROWS_T)])


_edge_split = pl.kernel(
    _edge_split_body,
    out_type=jax.ShapeDtypeStruct((NCORE, NP_, 64), jnp.float32),
    mesh=_sc_mesh(),
    compiler_params=pltpu.CompilerParams(use_tc_tiling_on_sc=False),
    scratch_types=[
        pltpu.VMEM((NCH2, K), jnp.int32),
        pltpu.VMEM((NCH2, K), jnp.int32),
        pltpu.VMEM((2, K, 64), jnp.float32),
        pltpu.VMEM_SHARED((NP_, 64), jnp.float32),
        pltpu.SemaphoreType.DMA,
        pltpu.SemaphoreType.DMA,
    ],
)


def _edge16_body(y_hbm, src_hbm, dst_hbm, acc_hbm, src_v, dst_v, rows_v,
                 acc_sh, g0, g1):
    c = lax.axis_index("c")
    s = lax.axis_index("s")
    w = s * NCORE + c
    for q in range(ROWS_T // K):
        pltpu.sync_copy(y_hbm.at[pl.ds(NN, K)],
                        acc_sh.at[pl.ds(s * ROWS_T + q * K, K)])
    pltpu.sync_copy(src_hbm.at[w], src_v)
    pltpu.sync_copy(dst_hbm.at[w], dst_v)
    plsc.subcore_barrier()

    def fire(j, b):
        sem = g0 if b == 0 else g1
        pltpu.async_copy(y_hbm.at[src_v.at[j]], rows_v.at[b], sem)

    def wait(b):
        sem = g0 if b == 0 else g1
        pltpu.make_async_copy(y_hbm.at[pl.ds(0, K)], rows_v.at[b], sem).wait()

    def scat(j, b):
        pltpu.sync_copy(rows_v.at[b], acc_sh.at[dst_v.at[j]], add=True)

    fire(0, 0)
    fire(1, 1)

    def body_i(i, carry):
        j0 = 2 * i
        wait(0)
        scat(j0, 0)
        fire(jnp.minimum(j0 + 2, NCH - 1), 0)
        wait(1)
        scat(j0 + 1, 1)
        fire(jnp.minimum(j0 + 3, NCH - 1), 1)
        return carry

    lax.fori_loop(0, NCH // 2, body_i, 0)
    wait(0)
    wait(1)
    plsc.subcore_barrier()
    pltpu.sync_copy(acc_sh.at[pl.ds(s * ROWS_T, ROWS_T)],
                    acc_hbm.at[c, pl.ds(s * ROWS_T, ROWS_T)])


_edge16 = pl.kernel(
    _edge16_body,
    out_type=jax.ShapeDtypeStruct((NCORE, NP_, 16), jnp.float32),
    mesh=_sc_mesh(),
    compiler_params=pltpu.CompilerParams(use_tc_tiling_on_sc=False),
    scratch_types=[
        pltpu.VMEM((NCH, K), jnp.int32),
        pltpu.VMEM((NCH, K), jnp.int32),
        pltpu.VMEM((2, K, 16), jnp.float32),
        pltpu.VMEM_SHARED((NP_, 16), jnp.float32),
        pltpu.SemaphoreType.DMA,
        pltpu.SemaphoreType.DMA,
    ],
)


def _split_rows(y):
    return jnp.concatenate([y[:, :64], y[:, 64:]], axis=0)


def _t1_body(degt_ref, xp_ref, w_ref, y_ref, dinv_ref):
    row = lax.broadcasted_iota(jnp.int32, (NP_, 1), 0)
    deg = degt_ref[:, 0:1] + degt_ref[:, 1:2]
    deg = deg + jnp.where(row < NN, 1.0, 0.0)
    dinv = jnp.where(deg > 0, lax.rsqrt(deg), 0.0)
    xw = jnp.dot(xp_ref[...], w_ref[...], preferred_element_type=jnp.float32)
    y_ref[...] = _split_rows(dinv * xw)
    dinv_ref[...] = dinv


def _t1_call(degt, xp, w1):
    return pl.pallas_call(
        _t1_body,
        out_shape=(jax.ShapeDtypeStruct((2 * NP_, 64), jnp.float32),
                   jax.ShapeDtypeStruct((NP_, 1), jnp.float32)),
    )(degt, xp, w1)


def _t2_body(split_out, y_ref, acc_ref, dinv_ref, b_ref, g_ref, be_ref,
             w_ref, out_ref):
    row = lax.broadcasted_iota(jnp.int32, (NP_, 1), 0)
    mask = row < NN
    dinv = dinv_ref[...]
    y = jnp.concatenate([y_ref[:NP_], y_ref[NP_:]], axis=1)
    acc = jnp.concatenate([acc_ref[0], acc_ref[1]], axis=1)
    h = dinv * (y + acc) + b_ref[...]
    hm = jnp.where(mask, h, 0.0)
    mean = jnp.sum(hm, axis=0, keepdims=True) * (1.0 / NN)
    d = jnp.where(mask, h - mean, 0.0)
    var = jnp.sum(d * d, axis=0, keepdims=True) * (1.0 / NN)
    hn = (h - mean) * lax.rsqrt(var + EPSV) * g_ref[...] + be_ref[...]
    r = jnp.where(mask, jnp.maximum(hn, 0.0), 0.0)
    yn = dinv * jnp.dot(r, w_ref[...], preferred_element_type=jnp.float32)
    out_ref[...] = _split_rows(yn) if split_out else yn


def _t2_call(split_out, y, acc, dinv, b, g, be, w):
    dn = w.shape[1]
    oshape = (2 * NP_, 64) if split_out else (NP_, dn)
    return pl.pallas_call(
        functools.partial(_t2_body, split_out),
        out_shape=jax.ShapeDtypeStruct(oshape, jnp.float32),
    )(y, acc, dinv, b, g, be, w)


def _t3_body(y_ref, acc_ref, dinv_ref, b_ref, out_ref):
    o = dinv_ref[...] * (y_ref[...] + acc_ref[0] + acc_ref[1]) + b_ref[...]
    out_ref[...] = o[:NN, :NCLS]


def _t3_call(y, acc, dinv, b):
    return pl.pallas_call(
        _t3_body,
        out_shape=jax.ShapeDtypeStruct((NN, NCLS), jnp.float32),
    )(y, acc, dinv, b)


def kernel(x, edge_index, W1, b1, g1, be1, W2, b2, g2, be2, W3, b3):
    src = edge_index[0]
    dst = edge_index[1]
    pad = jnp.full((EPAD - EE,), NN, jnp.int32)
    srcf = jnp.concatenate([src, pad])
    dstf = jnp.concatenate([dst, pad])
    src_a = srcf.reshape(NWORK, NCH, K)
    dst_a = dstf.reshape(NWORK, NCH, K)
    src_b = jnp.stack([srcf, srcf + NP_]).reshape(NCORE, NSUB, NCH2, K)
    dst_b = dstf.reshape(NSUB, NCH2, K)
    xp = jnp.pad(x, ((0, NP_ - NN), (0, 0)))

    degp = _deg_call(dst_a)
    y1, dinv = _t1_call(degp.T, xp, W1)
    acc1 = _edge_split(y1, src_b, dst_b)
    y2 = _t2_call(True, y1, acc1, dinv, b1.reshape(1, -1), g1.reshape(1, -1),
                  be1.reshape(1, -1), W2)
    acc2 = _edge_split(y2, src_b, dst_b)
    w3p = jnp.pad(W3, ((0, 0), (0, 16 - NCLS)))
    y3 = _t2_call(False, y2, acc2, dinv, b2.reshape(1, -1), g2.reshape(1, -1),
                  be2.reshape(1, -1), w3p)
    acc3 = _edge16(y3, src_a, dst_a)
    out = _t3_call(y3, acc3, dinv,
                   jnp.pad(b3, (0, 16 - NCLS)).reshape(1, -1))
    return out


# speedup vs baseline: 28.0958x; 2.0432x over previous
import functools

import jax
import jax.numpy as jnp
from jax import lax
from jax.experimental import pallas as pl
from jax.experimental.pallas import tpu as pltpu
from jax.experimental.pallas import tpu_sc as plsc

NN = 10000
EE = 320000
HID = 128
NCLS = 10
EPSV = 1e-5

NP_ = 10240
NCORE = 2
NSUB = 16
NWORK = NCORE * NSUB
K = 128
NCH = 80
NCH2 = 160
EPAD = NWORK * NCH * K
ROWS_T = NP_ // NSUB
NB = 4
PD = NB // 2


def _sc_mesh():
    return plsc.VectorSubcoreMesh(
        core_axis_name="c", subcore_axis_name="s",
        num_cores=NCORE, num_subcores=NSUB)


def _deg_body(dst_hbm, deg_hbm, dst_v, ones_v, z_v, deg_sh, sem):
    c = lax.axis_index("c")
    s = lax.axis_index("s")
    w = s * NCORE + c
    for i in range(K // 16):
        ones_v[pl.ds(i * 16, 16)] = jnp.ones((16,), jnp.float32)
    for i in range(ROWS_T // 16):
        z_v[pl.ds(i * 16, 16)] = jnp.zeros((16,), jnp.float32)
    pltpu.sync_copy(z_v, deg_sh.at[pl.ds(s * ROWS_T, ROWS_T)])
    pltpu.sync_copy(dst_hbm.at[w], dst_v)
    plsc.subcore_barrier()

    def fire(j, carry):
        pltpu.async_copy(ones_v, deg_sh.at[dst_v.at[j]], sem, add=True)
        return carry

    lax.fori_loop(0, NCH, fire, 0)

    def drain(j, carry):
        pltpu.make_async_copy(ones_v, deg_sh.at[pl.ds(0, K)], sem).wait()
        return carry

    lax.fori_loop(0, NCH, drain, 0)
    plsc.subcore_barrier()
    pltpu.sync_copy(deg_sh.at[pl.ds(s * ROWS_T, ROWS_T)],
                    deg_hbm.at[c, pl.ds(s * ROWS_T, ROWS_T)])


_deg_call = pl.kernel(
    _deg_body,
    out_type=jax.ShapeDtypeStruct((NCORE, NP_), jnp.float32),
    mesh=_sc_mesh(),
    compiler_params=pltpu.CompilerParams(use_tc_tiling_on_sc=False),
    scratch_types=[
        pltpu.VMEM((NCH, K), jnp.int32),
        pltpu.VMEM((K,), jnp.float32),
        pltpu.VMEM((ROWS_T,), jnp.float32),
        pltpu.VMEM_SHARED((NP_,), jnp.float32),
        pltpu.SemaphoreType.DMA,
    ],
)


def _make_edge(D, nch, feature_split):
    steady = nch - NB
    assert steady % NB == 0 and NB == 2 * PD

    def body(y_hbm, src_hbm, dst_hbm, acc_hbm, src_v, dst_v, rows_v, acc_sh,
             *sems):
        gs, ss = sems[:NB], sems[NB:]
        c = lax.axis_index("c")
        s = lax.axis_index("s")
        for q in range(ROWS_T // K):
            pltpu.sync_copy(y_hbm.at[pl.ds(NN, K)],
                            acc_sh.at[pl.ds(s * ROWS_T + q * K, K)])
        if feature_split:
            pltpu.sync_copy(src_hbm.at[c, s], src_v)
            pltpu.sync_copy(dst_hbm.at[s], dst_v)
        else:
            w = s * NCORE + c
            pltpu.sync_copy(src_hbm.at[w], src_v)
            pltpu.sync_copy(dst_hbm.at[w], dst_v)
        plsc.subcore_barrier()

        def fire_g(j, b):
            pltpu.async_copy(y_hbm.at[src_v.at[j]], rows_v.at[b], gs[b])

        def wait_g(b):
            pltpu.make_async_copy(y_hbm.at[pl.ds(0, K)], rows_v.at[b],
                                  gs[b]).wait()

        def fire_s(j, b):
            pltpu.async_copy(rows_v.at[b], acc_sh.at[dst_v.at[j]], ss[b],
                             add=True)

        def wait_s(b):
            pltpu.make_async_copy(rows_v.at[b], acc_sh.at[pl.ds(0, K)],
                                  ss[b]).wait()

        for j in range(PD):
            fire_g(j, j % NB)
        for j in range(PD):
            wait_g(j % NB)
            fire_s(j, j % NB)
            fire_g(j + PD, (j + PD) % NB)

        def sbody(i, carry):
            j0 = PD + NB * i
            for u in range(NB):
                b = (PD + u) % NB
                wait_g(b)
                fire_s(j0 + u, b)
                wait_s(u % NB)
                fire_g(j0 + u + PD, u % NB)
            return carry

        lax.fori_loop(0, steady // NB, sbody, 0)
        for t in range(PD):
            j = nch - PD + t
            wait_g(j % NB)
            fire_s(j, j % NB)
        for b in range(NB):
            wait_s(b)
        plsc.subcore_barrier()
        pltpu.sync_copy(acc_sh.at[pl.ds(s * ROWS_T, ROWS_T)],
                        acc_hbm.at[c, pl.ds(s * ROWS_T, ROWS_T)])

    return pl.kernel(
        body,
        out_type=jax.ShapeDtypeStruct((NCORE, NP_, D), jnp.float32),
        mesh=_sc_mesh(),
        compiler_params=pltpu.CompilerParams(use_tc_tiling_on_sc=False),
        scratch_types=[
            pltpu.VMEM((nch, K), jnp.int32),
            pltpu.VMEM((nch, K), jnp.int32),
            pltpu.VMEM((NB, K, D), jnp.float32),
            pltpu.VMEM_SHARED((NP_, D), jnp.float32),
        ] + [pltpu.SemaphoreType.DMA] * (2 * NB),
    )


_edge_split = _make_edge(64, NCH2, True)
_edge16 = _make_edge(16, NCH, False)


def _split_rows(y):
    return jnp.concatenate([y[:, :64], y[:, 64:]], axis=0)


def _t1_body(degt_ref, xp_ref, w_ref, y_ref, dinv_ref):
    row = lax.broadcasted_iota(jnp.int32, (NP_, 1), 0)
    deg = degt_ref[:, 0:1] + degt_ref[:, 1:2]
    deg = deg + jnp.where(row < NN, 1.0, 0.0)
    dinv = jnp.where(deg > 0, lax.rsqrt(deg), 0.0)
    xw = jnp.dot(xp_ref[...], w_ref[...], preferred_element_type=jnp.float32)
    y_ref[...] = _split_rows(dinv * xw)
    dinv_ref[...] = dinv


def _t1_call(degt, xp, w1):
    return pl.pallas_call(
        _t1_body,
        out_shape=(jax.ShapeDtypeStruct((2 * NP_, 64), jnp.float32),
                   jax.ShapeDtypeStruct((NP_, 1), jnp.float32)),
    )(degt, xp, w1)


def _t2_body(split_out, y_ref, acc_ref, dinv_ref, b_ref, g_ref, be_ref,
             w_ref, out_ref):
    row = lax.broadcasted_iota(jnp.int32, (NP_, 1), 0)
    mask = row < NN
    dinv = dinv_ref[...]
    y = jnp.concatenate([y_ref[:NP_], y_ref[NP_:]], axis=1)
    acc = jnp.concatenate([acc_ref[0], acc_ref[1]], axis=1)
    h = dinv * (y + acc) + b_ref[...]
    hm = jnp.where(mask, h, 0.0)
    mean = jnp.sum(hm, axis=0, keepdims=True) * (1.0 / NN)
    d = jnp.where(mask, h - mean, 0.0)
    var = jnp.sum(d * d, axis=0, keepdims=True) * (1.0 / NN)
    hn = (h - mean) * lax.rsqrt(var + EPSV) * g_ref[...] + be_ref[...]
    r = jnp.where(mask, jnp.maximum(hn, 0.0), 0.0)
    yn = dinv * jnp.dot(r, w_ref[...], preferred_element_type=jnp.float32)
    out_ref[...] = _split_rows(yn) if split_out else yn


def _t2_call(split_out, y, acc, dinv, b, g, be, w):
    dn = w.shape[1]
    oshape = (2 * NP_, 64) if split_out else (NP_, dn)
    return pl.pallas_call(
        functools.partial(_t2_body, split_out),
        out_shape=jax.ShapeDtypeStruct(oshape, jnp.float32),
    )(y, acc, dinv, b, g, be, w)


def _t3_body(y_ref, acc_ref, dinv_ref, b_ref, out_ref):
    o = dinv_ref[...] * (y_ref[...] + acc_ref[0] + acc_ref[1]) + b_ref[...]
    out_ref[...] = o[:NN, :NCLS]


def _t3_call(y, acc, dinv, b):
    return pl.pallas_call(
        _t3_body,
        out_shape=jax.ShapeDtypeStruct((NN, NCLS), jnp.float32),
    )(y, acc, dinv, b)


def kernel(x, edge_index, W1, b1, g1, be1, W2, b2, g2, be2, W3, b3):
    src = edge_index[0]
    dst = edge_index[1]
    pad = NN + jnp.arange(EPAD - EE, dtype=jnp.int32) % (NP_ - NN)
    srcf = jnp.concatenate([src, pad])
    dstf = jnp.concatenate([dst, pad])
    src_a = srcf.reshape(NWORK, NCH, K)
    dst_a = dstf.reshape(NWORK, NCH, K)
    src_b = jnp.stack([srcf, srcf + NP_]).reshape(NCORE, NSUB, NCH2, K)
    dst_b = dstf.reshape(NSUB, NCH2, K)
    xp = jnp.pad(x, ((0, NP_ - NN), (0, 0)))

    degp = _deg_call(dst_a)
    y1, dinv = _t1_call(degp.T, xp, W1)
    acc1 = _edge_split(y1, src_b, dst_b)
    y2 = _t2_call(True, y1, acc1, dinv, b1.reshape(1, -1), g1.reshape(1, -1),
                  be1.reshape(1, -1), W2)
    acc2 = _edge_split(y2, src_b, dst_b)
    w3p = jnp.pad(W3, ((0, 0), (0, 16 - NCLS)))
    y3 = _t2_call(False, y2, acc2, dinv, b2.reshape(1, -1), g2.reshape(1, -1),
                  be2.reshape(1, -1), w3p)
    acc3 = _edge16(y3, src_a, dst_a)
    out = _t3_call(y3, acc3, dinv,
                   jnp.pad(b3, (0, 16 - NCLS)).reshape(1, -1))
    return out
